# baseline (speedup 1.0000x reference)
import functools

import jax
import jax.numpy as jnp
from jax.experimental import pallas as pl
from jax.experimental.pallas import tpu as pltpu


def _round_up(x, m):
    return (x + m - 1) // m * m


def _emit_tile_stats(h, h_ref, psum_ref, psq_ref):
    h_ref[...] = h.astype(jnp.bfloat16)
    psum_ref[...] = jnp.broadcast_to(
        jnp.sum(h, axis=0, keepdims=True), psum_ref.shape)
    psq_ref[...] = jnp.broadcast_to(
        jnp.sum(h * h, axis=0, keepdims=True), psq_ref.shape)


_SYM_T = 4
_SYM_BLOCKS = [(p, q) for p in range(_SYM_T) for q in range(p, _SYM_T)]
_FIRST_TOUCH = {}
_LAST_TOUCH = {}
for _s, (_p, _q) in enumerate(_SYM_BLOCKS):
    for _r in {_p, _q}:
        _FIRST_TOUCH.setdefault(_r, _s)
        _LAST_TOUCH[_r] = _s


def _sym_walk(i, xw, adj_ref, h_ref, psum_ref, psq_ref, acc_scr, tm):
    for s, (p, q) in enumerate(_SYM_BLOCKS):

        @pl.when(i == s)
        def _(s=s, p=p, q=q):
            a = adj_ref[...]
            rp = acc_scr[p * tm:(p + 1) * tm, :]
            contrib_p = jnp.dot(a, xw[q * tm:(q + 1) * tm, :],
                                preferred_element_type=jnp.float32)
            acc_scr[p * tm:(p + 1) * tm, :] = (
                contrib_p if _FIRST_TOUCH[p] == s else rp + contrib_p)
            if p != q:
                rq = acc_scr[q * tm:(q + 1) * tm, :]
                contrib_q = jax.lax.dot_general(
                    a, xw[p * tm:(p + 1) * tm, :], (((0,), (0,)), ((), ())),
                    preferred_element_type=jnp.float32)
                acc_scr[q * tm:(q + 1) * tm, :] = (
                    contrib_q if _FIRST_TOUCH[q] == s else rq + contrib_q)
            for r in sorted({p, q}):
                if _LAST_TOUCH[r] == s:
                    _emit_tile_stats(acc_scr[r * tm:(r + 1) * tm, :],
                                     h_ref, psum_ref, psq_ref)


def _l1_body(xw_ref, adj_ref, h_ref, psum_ref, psq_ref, acc_scr, *, tm):
    i = pl.program_id(0)
    _sym_walk(i, xw_ref[...], adj_ref, h_ref, psum_ref, psq_ref, acc_scr, tm)


def _bn_finalize(ps, pq, gamma, beta, inv_n):
    total = jnp.sum(ps, axis=0, keepdims=True) * 0.125
    total_sq = jnp.sum(pq, axis=0, keepdims=True) * 0.125
    mean = total * inv_n
    var = jnp.maximum(total_sq * inv_n - mean * mean, 0.0)
    inv_std = jax.lax.rsqrt(var + 1e-5)
    scale = gamma * inv_std
    shift = beta - mean * scale
    return scale, shift


def _l2_body(h1_ref, ps_ref, pq_ref, g_ref, b_ref, w_ref, adj_ref,
             h_ref, psum_ref, psq_ref, xw_scr, acc_scr, *, tm, inv_n):
    i = pl.program_id(0)

    @pl.when(i == 0)
    def _():
        scale, shift = _bn_finalize(ps_ref[...], pq_ref[...], g_ref[...],
                                    b_ref[...], inv_n)
        a1 = jnp.maximum(
            h1_ref[...].astype(jnp.float32) * scale + shift, 0.0)
        xw_scr[...] = jnp.dot(
            a1.astype(jnp.bfloat16), w_ref[...].astype(jnp.bfloat16),
            preferred_element_type=jnp.float32).astype(jnp.bfloat16)

    _sym_walk(i, xw_scr[...], adj_ref, h_ref, psum_ref, psq_ref, acc_scr, tm)


def _bn_out_body(h_ref, ps_ref, pq_ref, g_ref, b_ref, out_ref, *, inv_n):
    scale, shift = _bn_finalize(ps_ref[...], pq_ref[...], g_ref[...],
                                b_ref[...], inv_n)
    y = h_ref[...].astype(jnp.float32) * scale + shift
    out_ref[...] = jnp.maximum(y, 0.0)


_ROWSTARTS = []
for _r in range(_SYM_T):
    _ROWSTARTS.append(_SYM_BLOCKS.index((_r, _r)))


def _walk_p(i):
    p = jnp.int32(0)
    for rs in _ROWSTARTS[1:]:
        p = p + (i >= rs).astype(jnp.int32)
    return p


def _adj_index(i):
    p = _walk_p(i)
    start = (p * (2 * _SYM_T - p + 1)) // 2
    return (p, i - start + p)


def _row_index(i):
    return (_walk_p(i), 0)


def _propagate1(xw1, adj_pad):
    n_pad = adj_pad.shape[0]
    f_pad = xw1.shape[1]
    tm = n_pad // _SYM_T
    body = functools.partial(_l1_body, tm=tm)
    return pl.pallas_call(
        body,
        out_shape=(
            jax.ShapeDtypeStruct((n_pad, f_pad), jnp.bfloat16),
            jax.ShapeDtypeStruct((_SYM_T * 8, f_pad), jnp.float32),
            jax.ShapeDtypeStruct((_SYM_T * 8, f_pad), jnp.float32),
        ),
        grid=(len(_SYM_BLOCKS),),
        in_specs=[
            pl.BlockSpec((n_pad, f_pad), lambda i: (0, 0)),
            pl.BlockSpec((tm, tm), _adj_index),
        ],
        out_specs=(
            pl.BlockSpec((tm, f_pad), _row_index),
            pl.BlockSpec((8, f_pad), _row_index),
            pl.BlockSpec((8, f_pad), _row_index),
        ),
        scratch_shapes=[pltpu.VMEM((n_pad, f_pad), jnp.float32)],
        compiler_params=pltpu.CompilerParams(
            dimension_semantics=("arbitrary",),
            vmem_limit_bytes=48 * 1024 * 1024),
    )(xw1, adj_pad)


def _propagate2(h1, ps1, pq1, g1, b1, w2p, adj_pad, n_real):
    n_pad = adj_pad.shape[0]
    f_in = h1.shape[1]
    f_pad = w2p.shape[1]
    tm = n_pad // _SYM_T
    body = functools.partial(_l2_body, tm=tm, inv_n=1.0 / n_real)
    return pl.pallas_call(
        body,
        out_shape=(
            jax.ShapeDtypeStruct((n_pad, f_pad), jnp.bfloat16),
            jax.ShapeDtypeStruct((_SYM_T * 8, f_pad), jnp.float32),
            jax.ShapeDtypeStruct((_SYM_T * 8, f_pad), jnp.float32),
        ),
        grid=(len(_SYM_BLOCKS),),
        in_specs=[
            pl.BlockSpec((n_pad, f_in), lambda i: (0, 0)),
            pl.BlockSpec(ps1.shape, lambda i: (0, 0)),
            pl.BlockSpec(pq1.shape, lambda i: (0, 0)),
            pl.BlockSpec((1, f_in), lambda i: (0, 0)),
            pl.BlockSpec((1, f_in), lambda i: (0, 0)),
            pl.BlockSpec((f_in, f_pad), lambda i: (0, 0)),
            pl.BlockSpec((tm, tm), _adj_index),
        ],
        out_specs=(
            pl.BlockSpec((tm, f_pad), _row_index),
            pl.BlockSpec((8, f_pad), _row_index),
            pl.BlockSpec((8, f_pad), _row_index),
        ),
        scratch_shapes=[pltpu.VMEM((n_pad, f_pad), jnp.bfloat16),
                        pltpu.VMEM((n_pad, f_pad), jnp.float32)],
        compiler_params=pltpu.CompilerParams(
            dimension_semantics=("arbitrary",),
            vmem_limit_bytes=48 * 1024 * 1024),
    )(h1, ps1, pq1, g1, b1, w2p, adj_pad)


def _bn_out(h2, ps2, pq2, g2, b2, n_real):
    n_pad, f_pad = h2.shape
    tm = n_pad // 2 if n_pad % 2 == 0 and n_pad >= 256 else n_pad
    m_tiles = n_pad // tm
    body = functools.partial(_bn_out_body, inv_n=1.0 / n_real)
    return pl.pallas_call(
        body,
        out_shape=jax.ShapeDtypeStruct((n_pad, f_pad), jnp.float32),
        grid=(m_tiles,),
        in_specs=[
            pl.BlockSpec((tm, f_pad), lambda i: (i, 0)),
            pl.BlockSpec(ps2.shape, lambda i: (0, 0)),
            pl.BlockSpec(pq2.shape, lambda i: (0, 0)),
            pl.BlockSpec((1, f_pad), lambda i: (0, 0)),
            pl.BlockSpec((1, f_pad), lambda i: (0, 0)),
        ],
        out_specs=pl.BlockSpec((tm, f_pad), lambda i: (i, 0)),
        compiler_params=pltpu.CompilerParams(
            dimension_semantics=("parallel",),
            vmem_limit_bytes=32 * 1024 * 1024),
    )(h2, ps2, pq2, g2, b2)


@functools.partial(jax.jit, static_argnames=("num_nodes",))
def _forward(w1, gamma1, beta1, w2, gamma2, beta2, x, adj_pad, num_nodes):
    n = num_nodes
    n_pad = adj_pad.shape[0]
    in_dim = x.shape[1]
    h_dim = w1.shape[1]
    out_dim = w2.shape[1]
    f1_pad = _round_up(h_dim, 128)
    f2_pad = _round_up(out_dim, 128)

    def pad_cols(v, f_pad):
        if v.shape[-1] == f_pad:
            return v.reshape(1, f_pad)
        return jnp.zeros((1, f_pad), jnp.float32).at[:, :v.shape[-1]].set(
            v.reshape(1, -1))

    x_pad = x
    if n_pad != n:
        x_pad = jnp.zeros((n_pad, in_dim), x.dtype).at[:n].set(x)

    w1p = w1
    if h_dim != f1_pad:
        w1p = jnp.zeros((in_dim, f1_pad), jnp.float32).at[:, :h_dim].set(w1)
    w2p = w2
    if h_dim != f1_pad or out_dim != f2_pad:
        w2p = jnp.zeros((f1_pad, f2_pad), jnp.float32)
        w2p = w2p.at[:h_dim, :out_dim].set(w2)

    xw1 = jnp.dot(x_pad.astype(jnp.bfloat16), w1p.astype(jnp.bfloat16),
                  preferred_element_type=jnp.float32).astype(jnp.bfloat16)
    h1, ps1, pq1 = _propagate1(xw1, adj_pad)
    h2, ps2, pq2 = _propagate2(
        h1, ps1, pq1, pad_cols(gamma1, f1_pad), pad_cols(beta1, f1_pad),
        w2p, adj_pad, n)
    out = _bn_out(h2, ps2, pq2, pad_cols(gamma2, f2_pad),
                  pad_cols(beta2, f2_pad), n)
    if n_pad != n or f2_pad != out_dim:
        out = out[:n, :out_dim]
    return out


def kernel(w1, b1, gamma1, beta1, w2, b2, gamma2, beta2, x, adj_pad):
    return _forward(w1, gamma1, beta1, w2, gamma2, beta2, x, adj_pad,
                    num_nodes=x.shape[0])


# speedup vs baseline: 1.0605x; 1.0605x over previous
import functools

import jax
import jax.numpy as jnp
from jax.experimental import pallas as pl
from jax.experimental.pallas import tpu as pltpu


def _round_up(x, m):
    return (x + m - 1) // m * m


def _emit_tile_stats(h, h_ref, psum_ref, psq_ref):
    h_ref[...] = h.astype(jnp.bfloat16)
    psum_ref[...] = jnp.broadcast_to(
        jnp.sum(h, axis=0, keepdims=True), psum_ref.shape)
    psq_ref[...] = jnp.broadcast_to(
        jnp.sum(h * h, axis=0, keepdims=True), psq_ref.shape)


def _sym_walk(i, xw, adj_ref, h_ref, psum_ref, psq_ref, acc_scr, tm):
    tc = 2 * tm

    def dot(a, b):
        return jnp.dot(a, b, preferred_element_type=jnp.float32)

    def dot_ta(a, b):
        return jax.lax.dot_general(a, b, (((0,), (0,)), ((), ())),
                                   preferred_element_type=jnp.float32)

    @pl.when(i == 0)
    def _():
        acc_scr[0:tm, :] = dot(adj_ref[...], xw[0:tc, :])

    @pl.when(i == 1)
    def _():
        acc_scr[tm:2 * tm, :] = dot(adj_ref[...], xw[0:tc, :])

    @pl.when(i == 2)
    def _():
        a = adj_ref[...]
        acc_scr[0:tm, :] += dot(a, xw[tc:, :])
        acc_scr[2 * tm:, :] = dot_ta(a, xw[0:tm, :])
        _emit_tile_stats(acc_scr[0:tm, :], h_ref, psum_ref, psq_ref)

    @pl.when(i == 3)
    def _():
        a = adj_ref[...]
        acc_scr[tm:2 * tm, :] += dot(a, xw[tc:, :])
        acc_scr[2 * tm:, :] += dot_ta(a, xw[tm:2 * tm, :])
        _emit_tile_stats(acc_scr[tm:2 * tm, :], h_ref, psum_ref, psq_ref)

    @pl.when(i == 4)
    def _():
        acc_scr[2 * tm:3 * tm, :] += dot(adj_ref[...], xw[tc:, :])
        _emit_tile_stats(acc_scr[2 * tm:3 * tm, :], h_ref, psum_ref, psq_ref)

    @pl.when(i == 5)
    def _():
        acc_scr[3 * tm:, :] += dot(adj_ref[...], xw[tc:, :])
        _emit_tile_stats(acc_scr[3 * tm:, :], h_ref, psum_ref, psq_ref)


def _l1_body(xw_ref, adj_ref, h_ref, psum_ref, psq_ref, acc_scr, *, tm):
    i = pl.program_id(0)
    _sym_walk(i, xw_ref[...], adj_ref, h_ref, psum_ref, psq_ref, acc_scr, tm)


def _bn_finalize(ps, pq, gamma, beta, inv_n):
    total = jnp.sum(ps, axis=0, keepdims=True) * 0.125
    total_sq = jnp.sum(pq, axis=0, keepdims=True) * 0.125
    mean = total * inv_n
    var = jnp.maximum(total_sq * inv_n - mean * mean, 0.0)
    inv_std = jax.lax.rsqrt(var + 1e-5)
    scale = gamma * inv_std
    shift = beta - mean * scale
    return scale, shift


def _l2_body(h1_ref, ps_ref, pq_ref, g_ref, b_ref, w_ref, adj_ref,
             h_ref, psum_ref, psq_ref, xw_scr, acc_scr, *, tm, inv_n):
    i = pl.program_id(0)

    @pl.when(i == 0)
    def _():
        scale, shift = _bn_finalize(ps_ref[...], pq_ref[...], g_ref[...],
                                    b_ref[...], inv_n)
        a1 = jnp.maximum(
            h1_ref[...].astype(jnp.float32) * scale + shift, 0.0)
        xw_scr[...] = jnp.dot(
            a1.astype(jnp.bfloat16), w_ref[...].astype(jnp.bfloat16),
            preferred_element_type=jnp.float32).astype(jnp.bfloat16)

    _sym_walk(i, xw_scr[...], adj_ref, h_ref, psum_ref, psq_ref, acc_scr, tm)


def _bn_out_body(h_ref, ps_ref, pq_ref, g_ref, b_ref, out_ref, *, inv_n):
    scale, shift = _bn_finalize(ps_ref[...], pq_ref[...], g_ref[...],
                                b_ref[...], inv_n)
    y = h_ref[...].astype(jnp.float32) * scale + shift
    out_ref[...] = jnp.maximum(y, 0.0)


def _adj_index(i):
    two = (i >= 2).astype(jnp.int32)
    return (i - 2 * two, two)


def _row_index(i):
    return (jnp.maximum(i - 2, 0), 0)


def _propagate1(xw1, adj_pad):
    n_pad = adj_pad.shape[0]
    f_pad = xw1.shape[1]
    tm = n_pad // 4
    body = functools.partial(_l1_body, tm=tm)
    return pl.pallas_call(
        body,
        out_shape=(
            jax.ShapeDtypeStruct((n_pad, f_pad), jnp.bfloat16),
            jax.ShapeDtypeStruct((32, f_pad), jnp.float32),
            jax.ShapeDtypeStruct((32, f_pad), jnp.float32),
        ),
        grid=(6,),
        in_specs=[
            pl.BlockSpec((n_pad, f_pad), lambda i: (0, 0)),
            pl.BlockSpec((tm, 2 * tm), _adj_index),
        ],
        out_specs=(
            pl.BlockSpec((tm, f_pad), _row_index),
            pl.BlockSpec((8, f_pad), _row_index),
            pl.BlockSpec((8, f_pad), _row_index),
        ),
        scratch_shapes=[pltpu.VMEM((n_pad, f_pad), jnp.float32)],
        compiler_params=pltpu.CompilerParams(
            dimension_semantics=("arbitrary",),
            vmem_limit_bytes=48 * 1024 * 1024),
    )(xw1, adj_pad)


def _propagate2(h1, ps1, pq1, g1, b1, w2p, adj_pad, n_real):
    n_pad = adj_pad.shape[0]
    f_in = h1.shape[1]
    f_pad = w2p.shape[1]
    tm = n_pad // 4
    body = functools.partial(_l2_body, tm=tm, inv_n=1.0 / n_real)
    return pl.pallas_call(
        body,
        out_shape=(
            jax.ShapeDtypeStruct((n_pad, f_pad), jnp.bfloat16),
            jax.ShapeDtypeStruct((32, f_pad), jnp.float32),
            jax.ShapeDtypeStruct((32, f_pad), jnp.float32),
        ),
        grid=(6,),
        in_specs=[
            pl.BlockSpec((n_pad, f_in), lambda i: (0, 0)),
            pl.BlockSpec(ps1.shape, lambda i: (0, 0)),
            pl.BlockSpec(pq1.shape, lambda i: (0, 0)),
            pl.BlockSpec((1, f_in), lambda i: (0, 0)),
            pl.BlockSpec((1, f_in), lambda i: (0, 0)),
            pl.BlockSpec((f_in, f_pad), lambda i: (0, 0)),
            pl.BlockSpec((tm, 2 * tm), _adj_index),
        ],
        out_specs=(
            pl.BlockSpec((tm, f_pad), _row_index),
            pl.BlockSpec((8, f_pad), _row_index),
            pl.BlockSpec((8, f_pad), _row_index),
        ),
        scratch_shapes=[pltpu.VMEM((n_pad, f_pad), jnp.bfloat16),
                        pltpu.VMEM((n_pad, f_pad), jnp.float32)],
        compiler_params=pltpu.CompilerParams(
            dimension_semantics=("arbitrary",),
            vmem_limit_bytes=48 * 1024 * 1024),
    )(h1, ps1, pq1, g1, b1, w2p, adj_pad)


def _bn_out(h2, ps2, pq2, g2, b2, n_real):
    n_pad, f_pad = h2.shape
    tm = n_pad // 2 if n_pad % 2 == 0 and n_pad >= 256 else n_pad
    m_tiles = n_pad // tm
    body = functools.partial(_bn_out_body, inv_n=1.0 / n_real)
    return pl.pallas_call(
        body,
        out_shape=jax.ShapeDtypeStruct((n_pad, f_pad), jnp.float32),
        grid=(m_tiles,),
        in_specs=[
            pl.BlockSpec((tm, f_pad), lambda i: (i, 0)),
            pl.BlockSpec(ps2.shape, lambda i: (0, 0)),
            pl.BlockSpec(pq2.shape, lambda i: (0, 0)),
            pl.BlockSpec((1, f_pad), lambda i: (0, 0)),
            pl.BlockSpec((1, f_pad), lambda i: (0, 0)),
        ],
        out_specs=pl.BlockSpec((tm, f_pad), lambda i: (i, 0)),
        compiler_params=pltpu.CompilerParams(
            dimension_semantics=("parallel",),
            vmem_limit_bytes=32 * 1024 * 1024),
    )(h2, ps2, pq2, g2, b2)


@functools.partial(jax.jit, static_argnames=("num_nodes",))
def _forward(w1, gamma1, beta1, w2, gamma2, beta2, x, adj_pad, num_nodes):
    n = num_nodes
    n_pad = adj_pad.shape[0]
    in_dim = x.shape[1]
    h_dim = w1.shape[1]
    out_dim = w2.shape[1]
    f1_pad = _round_up(h_dim, 128)
    f2_pad = _round_up(out_dim, 128)

    def pad_cols(v, f_pad):
        if v.shape[-1] == f_pad:
            return v.reshape(1, f_pad)
        return jnp.zeros((1, f_pad), jnp.float32).at[:, :v.shape[-1]].set(
            v.reshape(1, -1))

    x_pad = x
    if n_pad != n:
        x_pad = jnp.zeros((n_pad, in_dim), x.dtype).at[:n].set(x)

    w1p = w1
    if h_dim != f1_pad:
        w1p = jnp.zeros((in_dim, f1_pad), jnp.float32).at[:, :h_dim].set(w1)
    w2p = w2
    if h_dim != f1_pad or out_dim != f2_pad:
        w2p = jnp.zeros((f1_pad, f2_pad), jnp.float32)
        w2p = w2p.at[:h_dim, :out_dim].set(w2)

    xw1 = jnp.dot(x_pad.astype(jnp.bfloat16), w1p.astype(jnp.bfloat16),
                  preferred_element_type=jnp.float32).astype(jnp.bfloat16)
    h1, ps1, pq1 = _propagate1(xw1, adj_pad)
    h2, ps2, pq2 = _propagate2(
        h1, ps1, pq1, pad_cols(gamma1, f1_pad), pad_cols(beta1, f1_pad),
        w2p, adj_pad, n)
    out = _bn_out(h2, ps2, pq2, pad_cols(gamma2, f2_pad),
                  pad_cols(beta2, f2_pad), n)
    if n_pad != n or f2_pad != out_dim:
        out = out[:n, :out_dim]
    return out


def kernel(w1, b1, gamma1, beta1, w2, b2, gamma2, beta2, x, adj_pad):
    return _forward(w1, gamma1, beta1, w2, gamma2, beta2, x, adj_pad,
                    num_nodes=x.shape[0])


# speedup vs baseline: 1.2838x; 1.2105x over previous
import functools

import jax
import jax.numpy as jnp
from jax.experimental import pallas as pl
from jax.experimental.pallas import tpu as pltpu


def _round_up(x, m):
    return (x + m - 1) // m * m


def _finalize(psum_scr, psq_scr, gamma, beta, inv_n):
    total = jnp.sum(psum_scr[...], axis=0, keepdims=True) * 0.125
    total_sq = jnp.sum(psq_scr[...], axis=0, keepdims=True) * 0.125
    mean = total * inv_n
    var = jnp.maximum(total_sq * inv_n - mean * mean, 0.0)
    inv_std = jax.lax.rsqrt(var + 1e-5)
    scale = gamma * inv_std
    shift = beta - mean * scale
    return scale, shift


def _fused_body(xw1_ref, g1_ref, b1_ref, w2_ref, g2_ref, b2_ref, adj_ref,
                out_ref, acc_scr, xw_scr, psum_scr, psq_scr, *, tm, inv_n):
    i = pl.program_id(0)

    def dot(a, b):
        return jnp.dot(a, b, preferred_element_type=jnp.float32)

    def dot_ta(a, b):
        return jax.lax.dot_general(a, b, (((0,), (0,)), ((), ())),
                                   preferred_element_type=jnp.float32)

    def stats(h, first):
        ps = jnp.broadcast_to(jnp.sum(h, axis=0, keepdims=True),
                              psum_scr.shape)
        pq = jnp.broadcast_to(jnp.sum(h * h, axis=0, keepdims=True),
                              psq_scr.shape)
        if first:
            psum_scr[...] = ps
            psq_scr[...] = pq
        else:
            psum_scr[...] += ps
            psq_scr[...] += pq

    @pl.when(i == 0)
    def _():
        acc_scr[0:tm, :] = dot(adj_ref[...], xw1_ref[0:tm, :])

    @pl.when(i == 1)
    def _():
        a = adj_ref[...]
        acc_scr[0:tm, :] += dot(a, xw1_ref[tm:, :])
        acc_scr[tm:, :] = dot_ta(a, xw1_ref[0:tm, :])
        stats(acc_scr[0:tm, :], first=True)

    @pl.when(i == 2)
    def _():
        acc_scr[tm:, :] += dot(adj_ref[...], xw1_ref[tm:, :])
        stats(acc_scr[tm:, :], first=False)

    @pl.when(i == 3)
    def _():
        scale, shift = _finalize(psum_scr, psq_scr, g1_ref[...], b1_ref[...],
                                 inv_n)
        a1 = jnp.maximum(acc_scr[...] * scale + shift, 0.0)
        xw2 = dot(a1.astype(jnp.bfloat16),
                  w2_ref[...].astype(jnp.bfloat16)).astype(jnp.bfloat16)
        xw_scr[...] = xw2
        acc_scr[0:tm, :] = dot(adj_ref[...], xw2[0:tm, :])

    @pl.when(i == 4)
    def _():
        a = adj_ref[...]
        acc_scr[0:tm, :] += dot(a, xw_scr[tm:, :])
        acc_scr[tm:, :] = dot_ta(a, xw_scr[0:tm, :])
        stats(acc_scr[0:tm, :], first=True)

    @pl.when(i == 5)
    def _():
        acc_scr[tm:, :] += dot(adj_ref[...], xw_scr[tm:, :])
        stats(acc_scr[tm:, :], first=False)

    @pl.when(i == 6)
    def _():
        scale, shift = _finalize(psum_scr, psq_scr, g2_ref[...], b2_ref[...],
                                 inv_n)
        out_ref[...] = jnp.maximum(acc_scr[...] * scale + shift, 0.0)


def _adj_index(i):
    j = jnp.minimum(i - 3 * (i >= 3).astype(jnp.int32), 2)
    return (jnp.maximum(j - 1, 0), jnp.minimum(j, 1))


def _gcn_fused(xw1, g1, b1, w2p, g2, b2, adj_pad, n_real):
    n_pad = adj_pad.shape[0]
    f_pad = xw1.shape[1]
    tm = n_pad // 2
    body = functools.partial(_fused_body, tm=tm, inv_n=1.0 / n_real)
    return pl.pallas_call(
        body,
        out_shape=jax.ShapeDtypeStruct((n_pad, f_pad), jnp.float32),
        grid=(7,),
        in_specs=[
            pl.BlockSpec((n_pad, f_pad), lambda i: (0, 0)),
            pl.BlockSpec((1, f_pad), lambda i: (0, 0)),
            pl.BlockSpec((1, f_pad), lambda i: (0, 0)),
            pl.BlockSpec(w2p.shape, lambda i: (0, 0)),
            pl.BlockSpec((1, f_pad), lambda i: (0, 0)),
            pl.BlockSpec((1, f_pad), lambda i: (0, 0)),
            pl.BlockSpec((tm, tm), _adj_index),
        ],
        out_specs=pl.BlockSpec((n_pad, f_pad), lambda i: (0, 0)),
        scratch_shapes=[
            pltpu.VMEM((n_pad, f_pad), jnp.float32),
            pltpu.VMEM((n_pad, f_pad), jnp.bfloat16),
            pltpu.VMEM((8, f_pad), jnp.float32),
            pltpu.VMEM((8, f_pad), jnp.float32),
        ],
        compiler_params=pltpu.CompilerParams(
            dimension_semantics=("arbitrary",),
            vmem_limit_bytes=48 * 1024 * 1024),
    )(xw1, g1, b1, w2p, g2, b2, adj_pad)


@functools.partial(jax.jit, static_argnames=("num_nodes",))
def _forward(w1, gamma1, beta1, w2, gamma2, beta2, x, adj_pad, num_nodes):
    n = num_nodes
    n_pad = adj_pad.shape[0]
    in_dim = x.shape[1]
    h_dim = w1.shape[1]
    out_dim = w2.shape[1]
    f1_pad = _round_up(h_dim, 128)
    f2_pad = _round_up(out_dim, 128)

    def pad_cols(v, f_pad):
        if v.shape[-1] == f_pad:
            return v.reshape(1, f_pad)
        return jnp.zeros((1, f_pad), jnp.float32).at[:, :v.shape[-1]].set(
            v.reshape(1, -1))

    x_pad = x
    if n_pad != n:
        x_pad = jnp.zeros((n_pad, in_dim), x.dtype).at[:n].set(x)

    w1p = w1
    if h_dim != f1_pad:
        w1p = jnp.zeros((in_dim, f1_pad), jnp.float32).at[:, :h_dim].set(w1)
    w2p = w2
    if h_dim != f1_pad or out_dim != f2_pad:
        w2p = jnp.zeros((f1_pad, f2_pad), jnp.float32)
        w2p = w2p.at[:h_dim, :out_dim].set(w2)

    if f1_pad != f2_pad:
        raise NotImplementedError("fused path expects equal padded widths")

    xw1 = jnp.dot(x_pad.astype(jnp.bfloat16), w1p.astype(jnp.bfloat16),
                  preferred_element_type=jnp.float32).astype(jnp.bfloat16)
    out = _gcn_fused(xw1, pad_cols(gamma1, f1_pad), pad_cols(beta1, f1_pad),
                     w2p, pad_cols(gamma2, f2_pad), pad_cols(beta2, f2_pad),
                     adj_pad, n)
    if n_pad != n or f2_pad != out_dim:
        out = out[:n, :out_dim]
    return out


def kernel(w1, b1, gamma1, beta1, w2, b2, gamma2, beta2, x, adj_pad):
    return _forward(w1, gamma1, beta1, w2, gamma2, beta2, x, adj_pad,
                    num_nodes=x.shape[0])
